# Initial kernel scaffold; baseline (speedup 1.0000x reference)
import functools

import jax
import jax.numpy as jnp
from jax import lax
from jax.experimental import pallas as pl
from jax.experimental.pallas import tpu as pltpu
from jax.experimental.pallas import tpu_sc as plsc

N = 10000
D = 128
E = 160000
DP = 144
CH = 128
NCHUNK = E // CH
NCORES = 2
NSUB = 16
NW = NCORES * NSUB
CPW = -(-NCHUNK // NW)
RPS = N // NSUB


def _sc_scatter_body(feat_hbm, eidx_hbm, zeros_hbm, out_hbm,
                     src_v, dst_v, rows_v, acc_sh, sem):
    c = lax.axis_index("c")
    s = lax.axis_index("s")
    wid = c * NSUB + s

    pltpu.sync_copy(zeros_hbm.at[pl.ds(s * RPS, RPS)],
                    acc_sh.at[pl.ds(s * RPS, RPS)])
    plsc.subcore_barrier()

    def chunk(j, carry):
        cid = j * NW + wid

        @pl.when(cid < NCHUNK)
        def _():
            base = cid * CH
            pltpu.sync_copy(eidx_hbm.at[0, pl.ds(base, CH)], src_v)
            pltpu.sync_copy(eidx_hbm.at[1, pl.ds(base, CH)], dst_v)
            pltpu.async_copy(feat_hbm.at[src_v], rows_v, sem).wait()
            pltpu.sync_copy(rows_v, acc_sh.at[dst_v], add=True)

        return carry

    lax.fori_loop(0, CPW, chunk, 0)
    plsc.subcore_barrier()

    pltpu.sync_copy(acc_sh.at[pl.ds(s * RPS, RPS)],
                    out_hbm.at[c, pl.ds(s * RPS, RPS)])


_sc_scatter = functools.partial(
    pl.kernel,
    mesh=plsc.VectorSubcoreMesh(core_axis_name="c", subcore_axis_name="s",
                                num_cores=NCORES, num_subcores=NSUB),
    out_type=jax.ShapeDtypeStruct((NCORES, N, DP), jnp.float32),
    scratch_types=[
        pltpu.VMEM((CH,), jnp.int32),
        pltpu.VMEM((CH,), jnp.int32),
        pltpu.VMEM((CH, DP), jnp.float32),
        pltpu.VMEM_SHARED((N, DP), jnp.float32),
        pltpu.SemaphoreType.DMA,
    ],
)(_sc_scatter_body)


BLK = 1000


def _layer_body(p0_ref, p1_ref, xin_ref, wl_ref, wr_ref, b_ref, out_ref):
    S = p0_ref[...] + p1_ref[...]
    deg = jnp.maximum(S[:, D:D + 1], 1.0)
    agg = S[:, :D] / deg
    xin = xin_ref[:, :D]
    h = lax.dot_general(agg, wl_ref[...], (((1,), (1,)), ((), ())),
                        preferred_element_type=jnp.float32)
    h = h + lax.dot_general(xin, wr_ref[...], (((1,), (1,)), ((), ())),
                            preferred_element_type=jnp.float32)
    h = jnp.maximum(h + b_ref[...], 0.0)
    out_ref[:, :D] = h
    col = lax.broadcasted_iota(jnp.int32, (BLK, DP - D), 1)
    out_ref[:, D:DP] = jnp.where(col == 0, 1.0, 0.0)


def _layer(p0, p1, xin, wl, wr, b):
    return pl.pallas_call(
        _layer_body,
        grid=(N // BLK,),
        in_specs=[
            pl.BlockSpec((BLK, DP), lambda i: (i, 0)),
            pl.BlockSpec((BLK, DP), lambda i: (i, 0)),
            pl.BlockSpec((BLK, DP), lambda i: (i, 0)),
            pl.BlockSpec((D, D), lambda i: (0, 0)),
            pl.BlockSpec((D, D), lambda i: (0, 0)),
            pl.BlockSpec((1, D), lambda i: (0, 0)),
        ],
        out_specs=pl.BlockSpec((BLK, DP), lambda i: (i, 0)),
        out_shape=jax.ShapeDtypeStruct((N, DP), jnp.float32),
    )(p0, p1, xin, wl, wr, b)


BR = 200


def _decoder_body(zfull_ref, zblk_ref, out_ref):
    z = zfull_ref[:, :D]
    zb = zblk_ref[:, :D]
    logits = lax.dot_general(zb, z, (((1,), (1,)), ((), ())),
                             preferred_element_type=jnp.float32)
    m = jnp.max(logits, axis=1, keepdims=True)
    e = jnp.exp(logits - m)
    ssum = jnp.sum(e, axis=1, keepdims=True)
    out_ref[...] = e / ssum


def _decoder(z_pad):
    return pl.pallas_call(
        _decoder_body,
        grid=(N // BR,),
        in_specs=[
            pl.BlockSpec((N, DP), lambda i: (0, 0)),
            pl.BlockSpec((BR, DP), lambda i: (i, 0)),
        ],
        out_specs=pl.BlockSpec((BR, N), lambda i: (i, 0)),
        out_shape=jax.ShapeDtypeStruct((N, N), jnp.float32),
    )(z_pad, z_pad)


def kernel(x, edge_index, Wl1, Wr1, b1, Wl2, Wr2, b2):
    ones_col = jnp.concatenate(
        [jnp.ones((N, 1), jnp.float32), jnp.zeros((N, DP - D - 1), jnp.float32)],
        axis=1)
    x_pad = jnp.concatenate([x, ones_col], axis=1)
    zeros_pad = jnp.zeros((N, DP), jnp.float32)
    b1r = b1.reshape(1, D)
    b2r = b2.reshape(1, D)

    part1 = _sc_scatter(x_pad, edge_index, zeros_pad)
    h_pad = _layer(part1[0], part1[1], x_pad, Wl1, Wr1, b1r)
    part2 = _sc_scatter(h_pad, edge_index, zeros_pad)
    z_pad = _layer(part2[0], part2[1], h_pad, Wl2, Wr2, b2r)
    return _decoder(z_pad)


# speedup vs baseline: 9.0394x; 9.0394x over previous
import functools

import jax
import jax.numpy as jnp
from jax import lax
from jax.experimental import pallas as pl
from jax.experimental.pallas import tpu as pltpu
from jax.experimental.pallas import tpu_sc as plsc

N = 10000
D = 128
E = 160000
CH = 128
NCHUNK = E // CH
NCORES = 2
NSUB = 16
NW = NCORES * NSUB
CPW = -(-NCHUNK // NW)
RSTRIDE = 624
RWIN = 640


def _sc_scatter_body(feat_hbm, src_hbm, dst_hbm, zeros_hbm, zrow_hbm,
                     out_hbm, deg_hbm,
                     src_v, dst_v, rows_v, deg_v, acc_sh, sem):
    c = lax.axis_index("c")
    s = lax.axis_index("s")
    wid = c * NSUB + s

    pltpu.sync_copy(zeros_hbm.at[pl.ds(s * RSTRIDE, RWIN)],
                    acc_sh.at[pl.ds(s * RSTRIDE, RWIN)])
    pltpu.sync_copy(zrow_hbm, deg_v)
    plsc.subcore_barrier()

    zero16 = jnp.zeros((16,), jnp.int32)
    ones16 = jnp.ones((16,), jnp.float32)

    def chunk(j, carry):
        cid = j * NW + wid

        @pl.when(cid < NCHUNK)
        def _():
            base = cid * CH
            pltpu.sync_copy(src_hbm.at[pl.ds(base, CH)], src_v)
            pltpu.sync_copy(dst_hbm.at[pl.ds(base, CH)], dst_v)
            pltpu.async_copy(feat_hbm.at[src_v], rows_v, sem).wait()
            pltpu.sync_copy(rows_v, acc_sh.at[dst_v], add=True)
            for k in range(CH // 16):
                idx16 = dst_v[pl.ds(k * 16, 16)]
                plsc.addupdate_scatter(deg_v, [zero16, idx16], ones16)

        return carry

    lax.fori_loop(0, CPW, chunk, 0)
    plsc.subcore_barrier()

    pltpu.sync_copy(acc_sh.at[pl.ds(s * RSTRIDE, RWIN)],
                    out_hbm.at[c, pl.ds(s * RSTRIDE, RWIN)])
    pltpu.sync_copy(deg_v, deg_hbm.at[wid])


@functools.lru_cache(maxsize=1)
def _sc_scatter_kernel():
    return pl.kernel(
        _sc_scatter_body,
        mesh=plsc.VectorSubcoreMesh(core_axis_name="c", subcore_axis_name="s",
                                    num_cores=NCORES, num_subcores=NSUB),
        out_type=[
            jax.ShapeDtypeStruct((NCORES, N, D), jnp.float32),
            jax.ShapeDtypeStruct((NW, 1, N), jnp.float32),
        ],
        scratch_types=[
            pltpu.VMEM((CH,), jnp.int32),
            pltpu.VMEM((CH,), jnp.int32),
            pltpu.VMEM((CH, D), jnp.float32),
            pltpu.VMEM((1, N), jnp.float32),
            pltpu.VMEM_SHARED((N, D), jnp.float32),
            pltpu.SemaphoreType.DMA,
        ],
        compiler_params=pltpu.CompilerParams(needs_layout_passes=False),
    )


def _sc_scatter(feat, src, dst, zeros2d, zrow):
    return _sc_scatter_kernel()(feat, src, dst, zeros2d, zrow)


BLK = 1024


def _layer_body(p0_ref, p1_ref, degp_ref, xin_ref, wl_ref, wr_ref, b_ref,
                out_ref):
    S = p0_ref[...] + p1_ref[...]
    degp = degp_ref[...].reshape(NW, BLK)
    deg = lax.dot_general(degp, jnp.ones((NW, 1), jnp.float32),
                          (((0,), (0,)), ((), ())),
                          preferred_element_type=jnp.float32)
    agg = S / jnp.maximum(deg, 1.0)
    h = lax.dot_general(agg, wl_ref[...], (((1,), (1,)), ((), ())),
                        preferred_element_type=jnp.float32)
    h = h + lax.dot_general(xin_ref[...], wr_ref[...], (((1,), (1,)), ((), ())),
                            preferred_element_type=jnp.float32)
    out_ref[...] = jnp.maximum(h + b_ref[...], 0.0)


def _layer(p0, p1, degp, xin, wl, wr, b):
    return pl.pallas_call(
        _layer_body,
        grid=(-(-N // BLK),),
        in_specs=[
            pl.BlockSpec((BLK, D), lambda i: (i, 0)),
            pl.BlockSpec((BLK, D), lambda i: (i, 0)),
            pl.BlockSpec((NW, 1, BLK), lambda i: (0, 0, i)),
            pl.BlockSpec((BLK, D), lambda i: (i, 0)),
            pl.BlockSpec((D, D), lambda i: (0, 0)),
            pl.BlockSpec((D, D), lambda i: (0, 0)),
            pl.BlockSpec((1, D), lambda i: (0, 0)),
        ],
        out_specs=pl.BlockSpec((BLK, D), lambda i: (i, 0)),
        out_shape=jax.ShapeDtypeStruct((N, D), jnp.float32),
    )(p0, p1, degp, xin, wl, wr, b)


BR = 200


def _decoder_body(zfull_ref, zblk_ref, out_ref):
    logits = lax.dot_general(zblk_ref[...], zfull_ref[...],
                             (((1,), (1,)), ((), ())),
                             preferred_element_type=jnp.float32)
    m = jnp.max(logits, axis=1, keepdims=True)
    e = jnp.exp(logits - m)
    ssum = jnp.sum(e, axis=1, keepdims=True)
    out_ref[...] = e / ssum


def _decoder(z):
    return pl.pallas_call(
        _decoder_body,
        grid=(N // BR,),
        in_specs=[
            pl.BlockSpec((N, D), lambda i: (0, 0)),
            pl.BlockSpec((BR, D), lambda i: (i, 0)),
        ],
        out_specs=pl.BlockSpec((BR, N), lambda i: (i, 0)),
        out_shape=jax.ShapeDtypeStruct((N, N), jnp.float32),
    )(z, z)


def kernel(x, edge_index, Wl1, Wr1, b1, Wl2, Wr2, b2):
    src = edge_index[0]
    dst = edge_index[1]
    zeros2d = jnp.zeros((N, D), jnp.float32)
    zrow = jnp.zeros((1, N), jnp.float32)
    b1r = b1.reshape(1, D)
    b2r = b2.reshape(1, D)

    part1, degp = _sc_scatter(x, src, dst, zeros2d, zrow)
    h = _layer(part1[0], part1[1], degp, x, Wl1, Wr1, b1r)
    part2, degp2 = _sc_scatter(h, src, dst, zeros2d, zrow)
    z = _layer(part2[0], part2[1], degp2, h, Wl2, Wr2, b2r)
    return _decoder(z)
